# Initial kernel scaffold; baseline (speedup 1.0000x reference)
import jax
import jax.numpy as jnp
from jax.experimental import pallas as pl


def kernel(points, _inv, W_lin, W1, b1, W2, b2):
    raise NotImplementedError("write your pallas kernel here")


# speedup vs baseline: 1.7933x; 1.7933x over previous
import functools

import jax
import jax.numpy as jnp
from jax import lax
from jax.experimental import pallas as pl
from jax.experimental.pallas import tpu as pltpu
from jax.experimental.pallas import tpu_sc as plsc

N_POINTS = 320000
N_VOXELS = 10000
IN_CH = 128
OUT_CH = 128
HID = 4 * OUT_CH


ROWS_PER_BLOCK = 640
N_BLOCKS = N_POINTS // ROWS_PER_BLOCK


def _mlp_body(x_ref, wlt_ref, w1t_ref, b1_ref, w2t_ref, b2_ref, o_ref):
    x = x_ref[...]
    p = jnp.dot(x, wlt_ref[...], preferred_element_type=jnp.float32)
    h = jnp.dot(p, w1t_ref[...], preferred_element_type=jnp.float32)
    h = jnp.maximum(h + b1_ref[...], 0.0)
    a = jnp.dot(h, w2t_ref[...], preferred_element_type=jnp.float32)
    a = jax.nn.sigmoid(a + b2_ref[...])
    o_ref[...] = p * a


def _mlp_attend(points, wlt, w1t, b1r, w2t, b2r):
    return pl.pallas_call(
        _mlp_body,
        grid=(N_BLOCKS,),
        in_specs=[
            pl.BlockSpec((ROWS_PER_BLOCK, IN_CH), lambda i: (i, 0)),
            pl.BlockSpec((IN_CH, OUT_CH), lambda i: (0, 0)),
            pl.BlockSpec((OUT_CH, HID), lambda i: (0, 0)),
            pl.BlockSpec((1, HID), lambda i: (0, 0)),
            pl.BlockSpec((HID, OUT_CH), lambda i: (0, 0)),
            pl.BlockSpec((1, OUT_CH), lambda i: (0, 0)),
        ],
        out_specs=pl.BlockSpec((ROWS_PER_BLOCK, OUT_CH), lambda i: (i, 0)),
        out_shape=jax.ShapeDtypeStruct((N_POINTS, OUT_CH), jnp.float32),
    )(points, wlt, w1t, b1r, w2t, b2r)


CHUNK = 128
N_CHUNKS = N_POINTS // CHUNK
HALF = N_VOXELS // 2
ACC_ROWS = 5120
ZROWS = ACC_ROWS // 16
OUT_RC = 40
N_OUT_CHUNKS = HALF // OUT_RC

@functools.cache
def _make_segment_sum_sc():
    mesh = plsc.VectorSubcoreMesh(core_axis_name="c", subcore_axis_name="s")
    return functools.partial(
        pl.kernel,
        out_type=jax.ShapeDtypeStruct((N_VOXELS, OUT_CH), jnp.float32),
        mesh=mesh,
        scratch_types=[
            pltpu.VMEM((CHUNK,), jnp.int32),
            pltpu.VMEM((CHUNK,), jnp.int32),
            pltpu.VMEM((CHUNK, OUT_CH), jnp.float32),
            pltpu.VMEM((16,), jnp.int32),
            pltpu.VMEM((16,), jnp.int32),
            pltpu.VMEM((ZROWS, OUT_CH), jnp.float32),
            pltpu.VMEM_SHARED((ACC_ROWS, OUT_CH), jnp.float32),
        ],
    )(_segment_sum_body)


def _segment_sum_body(rows_hbm, inv_hbm, starts_hbm, ends_hbm, out_hbm,
                      idx_v, loc_v, rows_v, st_v, en_v, zbuf, acc):
    c = lax.axis_index("c")
    s = lax.axis_index("s")

    zeros16 = jnp.zeros((16,), jnp.float32)

    def _zero_row(r, _):
        for j in range(OUT_CH // 16):
            zbuf[r, pl.ds(j * 16, 16)] = zeros16
        return _

    lax.fori_loop(0, ZROWS, _zero_row, 0)
    pltpu.sync_copy(zbuf, acc.at[pl.ds(s * ZROWS, ZROWS)])
    plsc.subcore_barrier()

    pltpu.sync_copy(starts_hbm.at[c, s], st_v)
    pltpu.sync_copy(ends_hbm.at[c, s], en_v)
    st = st_v[...][0]
    en = en_v[...][0]

    base = c * HALF

    def _chunk(k, _):
        off = k * CHUNK
        pltpu.sync_copy(inv_hbm.at[pl.ds(off, CHUNK)], idx_v)
        pltpu.sync_copy(rows_hbm.at[pl.ds(off, CHUNK)], rows_v)
        for j in range(CHUNK // 16):
            iv = idx_v[pl.ds(j * 16, 16)]
            valid = (iv >= base) & (iv < base + HALF)
            loc_v[pl.ds(j * 16, 16)] = jnp.where(valid, iv - base, HALF)
        pltpu.sync_copy(rows_v, acc.at[loc_v], add=True)
        return _

    lax.fori_loop(st, en, _chunk, 0)
    plsc.subcore_barrier()

    def _out(i, carry):
        cid = s + i * 16

        @pl.when(cid < N_OUT_CHUNKS)
        def _copy_out():
            pltpu.sync_copy(
                acc.at[pl.ds(cid * OUT_RC, OUT_RC)],
                out_hbm.at[pl.ds(c * HALF + cid * OUT_RC, OUT_RC)],
            )

        return carry

    lax.fori_loop(0, (N_OUT_CHUNKS + 15) // 16, _out, 0)


def kernel(points, _inv, W_lin, W1, b1, W2, b2):
    inv32 = _inv.astype(jnp.int32)
    out_pts = _mlp_attend(
        points, W_lin.T, W1.T, b1.reshape(1, HID), W2.T, b2.reshape(1, OUT_CH)
    )

    split = jnp.searchsorted(inv32, HALF).astype(jnp.int32)
    cs0_end = (split + CHUNK - 1) // CHUNK
    cs1_start = split // CHUNK
    w = jnp.arange(16, dtype=jnp.int32)
    starts0 = w * cs0_end // 16
    ends0 = (w + 1) * cs0_end // 16
    n1 = N_CHUNKS - cs1_start
    starts1 = cs1_start + w * n1 // 16
    ends1 = cs1_start + (w + 1) * n1 // 16
    starts = jnp.broadcast_to(
        jnp.stack([starts0, starts1]).astype(jnp.int32)[:, :, None], (2, 16, 16)
    )
    ends = jnp.broadcast_to(
        jnp.stack([ends0, ends1]).astype(jnp.int32)[:, :, None], (2, 16, 16)
    )

    return _make_segment_sum_sc()(out_pts, inv32, starts, ends)
